# Initial kernel scaffold; baseline (speedup 1.0000x reference)
import jax
import jax.numpy as jnp
from jax.experimental import pallas as pl


def kernel(input, rois):
    raise NotImplementedError("write your pallas kernel here")


# speedup vs baseline: 3.6136x; 3.6136x over previous
import jax
import jax.numpy as jnp
from jax import lax
from jax.experimental import pallas as pl
from jax.experimental.pallas import tpu as pltpu
from jax.experimental.pallas import tpu_sc as plsc

H = 200
W = 200
C = 256
N = 2
R = 256
PH = 7
PW = 7
SCALE = 0.25
NC = 2
NS = 16
NW = NC * NS
RPW = R // NW
NTAP = 4 * 32
NBIN = PH * PW
OROI = NBIN * C


def _splat(v, dtype=jnp.float32):
    return jnp.broadcast_to(jnp.asarray(v, dtype), (16,))


def _body(feat, roisb, out, rois_v, xpart, idx_v, rows_v, ostage, sem0, sem1):
    cid = lax.axis_index("c")
    sid = lax.axis_index("s")
    wid = sid * NC + cid
    base_r = wid * RPW

    pltpu.sync_copy(roisb.at[pl.ds(base_r * 80, RPW * 80)], rois_v)

    lanes = lax.iota(jnp.int32, 16)
    lanesf = lanes.astype(jnp.float32)

    def fields(rl):
        b = rois_v[pl.ds(rl * 80, 16)]
        x1 = rois_v[pl.ds(rl * 80 + 16, 16)]
        y1 = rois_v[pl.ds(rl * 80 + 32, 16)]
        x2 = rois_v[pl.ds(rl * 80 + 48, 16)]
        y2 = rois_v[pl.ds(rl * 80 + 64, 16)]
        sx = x1 * SCALE - 0.5
        sy = y1 * SCALE - 0.5
        bw = (x2 * SCALE - 0.5 - sx) * (1.0 / PW)
        bh = (y2 * SCALE - 0.5 - sy) * (1.0 / PH)
        return b, sx, sy, bw, bh

    for i in range(RPW):
        b, sx, sy, bw, bh = fields(i)
        basev = b.astype(jnp.int32) * (H * W)
        xs = sx + (lanesf * 0.5 + 0.25) * bw
        xc = jnp.clip(xs, 0.0, float(W - 1))
        x0 = xc.astype(jnp.int32)
        x1 = jnp.minimum(x0 + 1, W - 1)
        xpart[pl.ds(i * 32, 16)] = basev + x0
        xpart[pl.ds(i * 32 + 16, 16)] = basev + x1

    def split_job(job):
        rl = job // PH
        ph = job - rl * PH
        return rl, ph

    def yrows(sy, bh, ph):
        phf = jnp.broadcast_to(ph.astype(jnp.float32), (16,))
        rows = []
        for iy in (0, 1):
            ys = sy + (phf + (0.25 + 0.5 * iy)) * bh
            yc = jnp.clip(ys, 0.0, float(H - 1))
            y0 = yc.astype(jnp.int32)
            y1 = jnp.minimum(y0 + 1, H - 1)
            rows.append(y0 * W)
            rows.append(y1 * W)
        return rows

    def build_and_fire(job, b, sem):
        rl, ph = split_job(job)
        _, sx, sy, bw, bh = fields(rl)
        yr = yrows(sy, bh, ph)
        xp0 = xpart[pl.ds(rl * 32, 16)]
        xp1 = xpart[pl.ds(rl * 32 + 16, 16)]
        for j in range(4):
            idx_v[b, pl.ds(j * 32, 16)] = yr[j] + xp0
            idx_v[b, pl.ds(j * 32 + 16, 16)] = yr[j] + xp1
        pltpu.async_copy(feat.at[idx_v.at[b]], rows_v.at[b], sem)

    def compute(job, b):
        rl, ph = split_job(job)
        _, sx, sy, bw, bh = fields(rl)
        phf = jnp.broadcast_to(ph.astype(jnp.float32), (16,))
        wy = []
        for iy in (0, 1):
            ys = sy + (phf + (0.25 + 0.5 * iy)) * bh
            my = jnp.where((ys >= -1.0) & (ys <= float(H)), 0.25, 0.0)
            yc = jnp.clip(ys, 0.0, float(H - 1))
            ly = yc - yc.astype(jnp.int32).astype(jnp.float32)
            wy.append((1.0 - ly) * my)
            wy.append(ly * my)

        def bin_body(pw, _):
            pwf = jnp.broadcast_to(pw.astype(jnp.float32), (16,))
            wx = [[None, None], [None, None]]
            for ix in (0, 1):
                xs = sx + (pwf + (0.25 + 0.5 * ix)) * bw
                mx = jnp.where((xs >= -1.0) & (xs <= float(W)), 1.0, 0.0)
                xc = jnp.clip(xs, 0.0, float(W - 1))
                lx = xc - xc.astype(jnp.int32).astype(jnp.float32)
                wx[0][ix] = (1.0 - lx) * mx
                wx[1][ix] = lx * mx
            wts = [wy[j] * wx[tx][ix]
                   for j in range(4) for tx in (0, 1) for ix in (0, 1)]
            s2 = pw * 2
            obase = (ph * PW + pw) * C
            for c in range(16):
                acc = jnp.zeros((16,), jnp.float32)
                t = 0
                for j in range(4):
                    for tx in (0, 1):
                        for ix in (0, 1):
                            u = j * 32 + tx * 16 + s2 + ix
                            acc = acc + wts[t] * rows_v[b, u, pl.ds(c * 16, 16)]
                            t += 1
                ostage[pl.ds(obase + c * 16, 16)] = acc
            return 0

        lax.fori_loop(0, PW, bin_body, 0, unroll=False)
        return rl, ph

    build_and_fire(jnp.int32(0), 0, sem0)
    build_and_fire(jnp.int32(1), 1, sem1)

    def it_body(it, _):
        for b, sem in ((0, sem0), (1, sem1)):
            job = it * 2 + b
            pltpu.make_async_copy(feat.at[idx_v.at[b]], rows_v.at[b],
                                  sem).wait()
            rl, ph = compute(job, b)

            @pl.when(ph == PH - 1)
            def _():
                pltpu.sync_copy(
                    ostage, out.at[pl.ds((base_r + rl) * OROI, OROI)])

            build_and_fire(jnp.minimum(job + 2, RPW * PH - 1), b, sem)
        return 0

    lax.fori_loop(0, RPW * PH // 2, it_body, 0, unroll=False)
    pltpu.make_async_copy(feat.at[idx_v.at[0]], rows_v.at[0], sem0).wait()
    pltpu.make_async_copy(feat.at[idx_v.at[1]], rows_v.at[1], sem1).wait()


@jax.jit
def _roialign_sc(feat_flat, rois_b):
    mesh = plsc.VectorSubcoreMesh(core_axis_name="c", subcore_axis_name="s")
    run = pl.kernel(
        _body,
        out_type=jax.ShapeDtypeStruct((R * OROI,), jnp.float32),
        mesh=mesh,
        scratch_types=[
            pltpu.VMEM((RPW * 80,), jnp.float32),
            pltpu.VMEM((RPW * 32,), jnp.int32),
            pltpu.VMEM((2, NTAP), jnp.int32),
            pltpu.VMEM((2, NTAP, C), jnp.float32),
            pltpu.VMEM((OROI,), jnp.float32),
            pltpu.SemaphoreType.DMA,
            pltpu.SemaphoreType.DMA,
        ],
    )
    return run(feat_flat, rois_b)


def kernel(input, rois):
    feat_flat = jnp.transpose(input, (0, 2, 3, 1)).reshape(N * H * W, C)
    rois_b = jnp.repeat(rois.reshape(-1), 16)
    out = _roialign_sc(feat_flat, rois_b)
    return jnp.transpose(out.reshape(R, PH, PW, C), (0, 3, 1, 2))
